# baseline (speedup 1.0000x reference)
import math
import functools

import jax
import jax.numpy as jnp
from jax import lax
from jax.experimental import pallas as pl
from jax.experimental.pallas import tpu as pltpu
from jax.experimental.pallas import tpu_sc as plsc

VOCAB = 100000
DIM = 64
BATCH = 4096
SEQ = 200
LANES = 16
BTILE = 128
DTILE = 8

_info = plsc.get_sparse_core_info()
NC, NS = _info.num_cores, _info.num_subcores
NW = NC * NS


def _pos_encoding():
    position = jnp.arange(SEQ, dtype=jnp.float32)[:, None]
    div_term = jnp.exp(
        jnp.arange(0, DIM, 2, dtype=jnp.float32) * (-math.log(10000.0) / DIM)
    )
    pe = jnp.zeros((SEQ, DIM), dtype=jnp.float32)
    pe = pe.at[:, 0::2].set(jnp.sin(position * div_term))
    pe = pe.at[:, 1::2].set(jnp.cos(position * div_term))
    return pe


def _transpose_block(rows, tr, pe_flat, s):
    iota = lax.broadcasted_iota(jnp.int32, (LANES,), 0)
    for c in range(DIM // LANES):
        pe_row = pe_flat[pl.ds(s * DIM + c * LANES, LANES)]

        @plsc.parallel_loop(0, LANES, unroll=4)
        def jstep(j, pe_row=pe_row, c=c):
            rot = (j + iota) & (LANES - 1)
            cidx = c * LANES + rot
            pe_rot = pe_row.at[rot].get(mode="promise_in_bounds")
            off_base = cidx * BTILE + iota
            for bt in range(BTILE // LANES):
                ridx = iota + (bt * LANES)
                v = plsc.load_gather(rows, [ridx, cidx])
                plsc.store_scatter(tr, [off_base + bt * LANES], v + pe_rot)


def _body(idx_hbm, table_hbm, pe_hbm, out_hbm, idx_v, pe_flat, rows0, rows1, tr0, tr1, sem_g, sem_o):
    c = lax.axis_index("c")
    s_ax = lax.axis_index("s")
    w = s_ax * NC + c
    pltpu.sync_copy(idx_hbm.at[w], idx_v)
    pltpu.sync_copy(pe_hbm, pe_flat)

    def gather_desc(s, rows, phase):
        return pltpu.make_async_copy(
            table_hbm.at[idx_v.at[s]], rows, sem_g.at[phase]
        )

    def out_descs(s, tr, phase):
        return [
            pltpu.make_async_copy(
                tr.at[pl.ds(g * DTILE * BTILE, DTILE * BTILE)],
                out_hbm.at[s, g, w],
                sem_o.at[phase],
            )
            for g in range(DIM // DTILE)
        ]

    gather_desc(0, rows0, 0).start()

    def step(s2, carry):
        for phase, rows, nrows, tr in (
            (0, rows0, rows1, tr0),
            (1, rows1, rows0, tr1),
        ):
            s = 2 * s2 + phase
            gather_desc(s, rows, phase).wait()

            @pl.when(s + 1 < SEQ)
            def _():
                gather_desc(s + 1, nrows, 1 - phase).start()

            @pl.when(s >= 2)
            def _():
                for d in out_descs(s - 2, tr, phase):
                    d.wait()

            _transpose_block(rows, tr, pe_flat, s)
            for d in out_descs(s, tr, phase):
                d.start()
        return carry

    lax.fori_loop(0, SEQ // 2, step, 0)

    for phase, tr in ((0, tr0), (1, tr1)):
        for d in out_descs(SEQ - 2 + phase, tr, phase):
            d.wait()


@jax.jit
def _run(idx, table, pe):
    mesh = plsc.VectorSubcoreMesh(core_axis_name="c", subcore_axis_name="s")
    k = pl.kernel(
        _body,
        out_type=jax.ShapeDtypeStruct(
            (SEQ, DIM // DTILE, NW, DTILE * BTILE), jnp.float32
        ),
        mesh=mesh,
        compiler_params=pltpu.CompilerParams(use_tc_tiling_on_sc=False, needs_layout_passes=False),
        scratch_types=[
            pltpu.VMEM((SEQ, BTILE), jnp.int32),
            pltpu.VMEM((SEQ * DIM,), jnp.float32),
            pltpu.VMEM((BTILE, DIM), jnp.float32),
            pltpu.VMEM((BTILE, DIM), jnp.float32),
            pltpu.VMEM((DIM * BTILE,), jnp.float32),
            pltpu.VMEM((DIM * BTILE,), jnp.float32),
            pltpu.SemaphoreType.DMA((2,)),
            pltpu.SemaphoreType.DMA((2,)),
        ],
    )
    return k(idx, table, pe)


def kernel(char_indices, embedding_weight):
    idx = (
        char_indices.astype(jnp.int32)
        .T.reshape(SEQ, NW, BTILE)
        .transpose(1, 0, 2)
    )
    pe = _pos_encoding().reshape(SEQ * DIM)
    x5 = _run(idx, embedding_weight, pe).reshape(SEQ, DIM // DTILE, NW, DTILE, BTILE)
    return jnp.transpose(x5, (2, 4, 0, 1, 3)).reshape(BATCH, SEQ, DIM)


# speedup vs baseline: 1.1534x; 1.1534x over previous
import math
import functools

import jax
import jax.numpy as jnp
from jax import lax
from jax.experimental import pallas as pl
from jax.experimental.pallas import tpu as pltpu
from jax.experimental.pallas import tpu_sc as plsc

VOCAB = 100000
DIM = 64
BATCH = 4096
SEQ = 200
LANES = 16
BTILE = 128
DTILE = 8

_info = plsc.get_sparse_core_info()
NC, NS = _info.num_cores, _info.num_subcores
NW = NC * NS


def _pos_encoding():
    position = jnp.arange(SEQ, dtype=jnp.float32)[:, None]
    div_term = jnp.exp(
        jnp.arange(0, DIM, 2, dtype=jnp.float32) * (-math.log(10000.0) / DIM)
    )
    pe = jnp.zeros((SEQ, DIM), dtype=jnp.float32)
    pe = pe.at[:, 0::2].set(jnp.sin(position * div_term))
    pe = pe.at[:, 1::2].set(jnp.cos(position * div_term))
    return pe


def _transpose_block(rows, tr, pe_flat, s):
    iota = lax.broadcasted_iota(jnp.int32, (LANES,), 0)

    @plsc.parallel_loop(0, DIM, unroll=2)
    def jstep(q):
        c = q >> 4
        j = q & (LANES - 1)
        pe_row = pe_flat[pl.ds(s * DIM + c * LANES, LANES)]
        rot = (j + iota) & (LANES - 1)
        cidx = c * LANES + rot
        pe_rot = pe_row.at[rot].get(mode="promise_in_bounds")
        off_base = cidx * BTILE + iota
        for bt in range(BTILE // LANES):
            ridx = iota + (bt * LANES)
            v = plsc.load_gather(rows, [ridx, cidx])
            plsc.store_scatter(tr, [off_base + bt * LANES], v + pe_rot)


def _body(idx_hbm, table_hbm, pe_hbm, out_hbm, idx_v, pe_flat, rows0, rows1, tr0, tr1, sem_g, sem_o):
    c = lax.axis_index("c")
    s_ax = lax.axis_index("s")
    w = s_ax * NC + c
    pltpu.sync_copy(idx_hbm.at[w], idx_v)
    pltpu.sync_copy(pe_hbm, pe_flat)

    def gather_desc(s, rows, phase):
        return pltpu.make_async_copy(
            table_hbm.at[idx_v.at[s]], rows, sem_g.at[phase]
        )

    def out_descs(s, tr, phase):
        return [
            pltpu.make_async_copy(
                tr.at[pl.ds(g * DTILE * BTILE, DTILE * BTILE)],
                out_hbm.at[s, g, w],
                sem_o.at[phase],
            )
            for g in range(DIM // DTILE)
        ]

    gather_desc(0, rows0, 0).start()

    def step(s2, carry):
        for phase, rows, nrows, tr in (
            (0, rows0, rows1, tr0),
            (1, rows1, rows0, tr1),
        ):
            s = 2 * s2 + phase
            gather_desc(s, rows, phase).wait()

            @pl.when(s + 1 < SEQ)
            def _():
                gather_desc(s + 1, nrows, 1 - phase).start()

            @pl.when(s >= 2)
            def _():
                for d in out_descs(s - 2, tr, phase):
                    d.wait()

            _transpose_block(rows, tr, pe_flat, s)
            for d in out_descs(s, tr, phase):
                d.start()
        return carry

    lax.fori_loop(0, SEQ // 2, step, 0)

    for phase, tr in ((0, tr0), (1, tr1)):
        for d in out_descs(SEQ - 2 + phase, tr, phase):
            d.wait()


@jax.jit
def _run(idx, table, pe):
    mesh = plsc.VectorSubcoreMesh(core_axis_name="c", subcore_axis_name="s")
    k = pl.kernel(
        _body,
        out_type=jax.ShapeDtypeStruct(
            (SEQ, DIM // DTILE, NW, DTILE * BTILE), jnp.float32
        ),
        mesh=mesh,
        compiler_params=pltpu.CompilerParams(use_tc_tiling_on_sc=False, needs_layout_passes=False),
        scratch_types=[
            pltpu.VMEM((SEQ, BTILE), jnp.int32),
            pltpu.VMEM((SEQ * DIM,), jnp.float32),
            pltpu.VMEM((BTILE, DIM), jnp.float32),
            pltpu.VMEM((BTILE, DIM), jnp.float32),
            pltpu.VMEM((DIM * BTILE,), jnp.float32),
            pltpu.VMEM((DIM * BTILE,), jnp.float32),
            pltpu.SemaphoreType.DMA((2,)),
            pltpu.SemaphoreType.DMA((2,)),
        ],
    )
    return k(idx, table, pe)


def kernel(char_indices, embedding_weight):
    idx = (
        char_indices.astype(jnp.int32)
        .T.reshape(SEQ, NW, BTILE)
        .transpose(1, 0, 2)
    )
    pe = _pos_encoding().reshape(SEQ * DIM)
    x5 = _run(idx, embedding_weight, pe).reshape(SEQ, DIM // DTILE, NW, DTILE, BTILE)
    return jnp.transpose(x5, (2, 4, 0, 1, 3)).reshape(BATCH, SEQ, DIM)


# speedup vs baseline: 1.1559x; 1.0022x over previous
import math
import functools

import jax
import jax.numpy as jnp
from jax import lax
from jax.experimental import pallas as pl
from jax.experimental.pallas import tpu as pltpu
from jax.experimental.pallas import tpu_sc as plsc

VOCAB = 100000
DIM = 64
BATCH = 4096
SEQ = 200
LANES = 16
BTILE = 128
DTILE = 8

_info = plsc.get_sparse_core_info()
NC, NS = _info.num_cores, _info.num_subcores
NW = NC * NS


def _pos_encoding():
    import numpy as np

    position = np.arange(SEQ, dtype=np.float64)[:, None]
    div_term = np.exp(
        np.arange(0, DIM, 2, dtype=np.float32).astype(np.float64)
        * (-math.log(10000.0) / DIM)
    )
    pe = np.zeros((SEQ, DIM), dtype=np.float32)
    pe[:, 0::2] = np.sin(position * div_term).astype(np.float32)
    pe[:, 1::2] = np.cos(position * div_term).astype(np.float32)
    return jnp.asarray(pe)


def _transpose_block(rows, tr, pe_flat, s):
    iota = lax.broadcasted_iota(jnp.int32, (LANES,), 0)

    @plsc.parallel_loop(0, DIM, unroll=2)
    def jstep(q):
        c = q >> 4
        j = q & (LANES - 1)
        pe_row = pe_flat[pl.ds(s * DIM + c * LANES, LANES)]
        rot = (j + iota) & (LANES - 1)
        cidx = c * LANES + rot
        pe_rot = pe_row.at[rot].get(mode="promise_in_bounds")
        off_base = cidx * BTILE + iota
        for bt in range(BTILE // LANES):
            ridx = iota + (bt * LANES)
            v = plsc.load_gather(rows, [ridx, cidx])
            plsc.store_scatter(tr, [off_base + bt * LANES], v + pe_rot)


def _body(idx_hbm, table_hbm, pe_hbm, out_hbm, idx_v, pe_flat, rows0, rows1, tr0, tr1, sem_g, sem_o):
    c = lax.axis_index("c")
    s_ax = lax.axis_index("s")
    w = s_ax * NC + c
    pltpu.sync_copy(idx_hbm.at[w], idx_v)
    pltpu.sync_copy(pe_hbm, pe_flat)

    def gather_desc(s, rows, phase):
        return pltpu.make_async_copy(
            table_hbm.at[idx_v.at[s]], rows, sem_g.at[phase]
        )

    def out_descs(s, tr, phase):
        return [
            pltpu.make_async_copy(
                tr.at[pl.ds(g * DTILE * BTILE, DTILE * BTILE)],
                out_hbm.at[s, g, w],
                sem_o.at[phase],
            )
            for g in range(DIM // DTILE)
        ]

    gather_desc(0, rows0, 0).start()

    def step(s2, carry):
        for phase, rows, nrows, tr in (
            (0, rows0, rows1, tr0),
            (1, rows1, rows0, tr1),
        ):
            s = 2 * s2 + phase
            gather_desc(s, rows, phase).wait()

            @pl.when(s + 1 < SEQ)
            def _():
                gather_desc(s + 1, nrows, 1 - phase).start()

            @pl.when(s >= 2)
            def _():
                for d in out_descs(s - 2, tr, phase):
                    d.wait()

            _transpose_block(rows, tr, pe_flat, s)
            for d in out_descs(s, tr, phase):
                d.start()
        return carry

    lax.fori_loop(0, SEQ // 2, step, 0)

    for phase, tr in ((0, tr0), (1, tr1)):
        for d in out_descs(SEQ - 2 + phase, tr, phase):
            d.wait()


@jax.jit
def _run(idx, table, pe):
    mesh = plsc.VectorSubcoreMesh(core_axis_name="c", subcore_axis_name="s")
    k = pl.kernel(
        _body,
        out_type=jax.ShapeDtypeStruct(
            (SEQ, DIM // DTILE, NW, DTILE * BTILE), jnp.float32
        ),
        mesh=mesh,
        compiler_params=pltpu.CompilerParams(use_tc_tiling_on_sc=False, needs_layout_passes=False),
        scratch_types=[
            pltpu.VMEM((SEQ, BTILE), jnp.int32),
            pltpu.VMEM((SEQ * DIM,), jnp.float32),
            pltpu.VMEM((BTILE, DIM), jnp.float32),
            pltpu.VMEM((BTILE, DIM), jnp.float32),
            pltpu.VMEM((DIM * BTILE,), jnp.float32),
            pltpu.VMEM((DIM * BTILE,), jnp.float32),
            pltpu.SemaphoreType.DMA((2,)),
            pltpu.SemaphoreType.DMA((2,)),
        ],
    )
    return k(idx, table, pe)


def kernel(char_indices, embedding_weight):
    idx = (
        char_indices.astype(jnp.int32)
        .T.reshape(SEQ, NW, BTILE)
        .transpose(1, 0, 2)
    )
    pe = _pos_encoding().reshape(SEQ * DIM)
    x5 = _run(idx, embedding_weight, pe).reshape(SEQ, DIM // DTILE, NW, DTILE, BTILE)
    return jnp.transpose(x5, (2, 4, 0, 1, 3)).reshape(BATCH, SEQ, DIM)
